# baseline (speedup 1.0000x reference)
import jax
import jax.numpy as jnp
from jax.experimental import pallas as pl
from jax.experimental.pallas import tpu as pltpu

_N = 10000
_FT = 128
_BM = 512


def _gcn_block_kernel(seq_ref, w_ref, b_ref, a_ref, adj_ref, out_ref, h_ref):
    i = pl.program_id(0)

    @pl.when(i == 0)
    def _compute_h():
        h = jax.lax.dot_general(
            seq_ref[...], w_ref[...], (((1,), (1,)), ((), ())),
            preferred_element_type=jnp.float32,
        ) + b_ref[...]
        h_ref[...] = h

    o = jnp.dot(adj_ref[...], h_ref[...], preferred_element_type=jnp.float32)
    alpha = a_ref[0, 0]
    out_ref[...] = jnp.where(o >= 0, o, alpha * o)


def kernel(seq, adj, W, b, a):
    seq2 = seq.reshape(_N, _FT)
    adj2 = adj.reshape(_N, _N)
    b2 = b.reshape(1, _FT)
    a2 = a.reshape(1, 1)

    out = pl.pallas_call(
        _gcn_block_kernel,
        grid=(pl.cdiv(_N, _BM),),
        in_specs=[
            pl.BlockSpec((_N, _FT), lambda i: (0, 0)),
            pl.BlockSpec((_FT, _FT), lambda i: (0, 0)),
            pl.BlockSpec((1, _FT), lambda i: (0, 0)),
            pl.BlockSpec((1, 1), lambda i: (0, 0)),
            pl.BlockSpec((_BM, _N), lambda i: (i, 0)),
        ],
        out_specs=pl.BlockSpec((_BM, _FT), lambda i: (i, 0)),
        out_shape=jax.ShapeDtypeStruct((_N, _FT), jnp.float32),
        scratch_shapes=[pltpu.VMEM((_N, _FT), jnp.float32)],
        compiler_params=pltpu.CompilerParams(vmem_limit_bytes=64 * 1024 * 1024),
    )(seq2, W, b2, a2, adj2)
    return out.reshape(1, _N, _FT)


# speedup vs baseline: 1.0097x; 1.0097x over previous
import jax
import jax.numpy as jnp
from jax.experimental import pallas as pl
from jax.experimental.pallas import tpu as pltpu

_N = 10000
_FT = 128
_S = 2
_BM = 200


def _gcn_block_kernel(seq_ref, w_ref, b_ref, a_ref, adj_ref, out_ref, h_ref):
    i = pl.program_id(0)

    @pl.when(i == 0)
    def _compute_h():
        h = jax.lax.dot_general(
            seq_ref[...], w_ref[...], (((1,), (1,)), ((), ())),
            preferred_element_type=jnp.float32,
        ) + b_ref[...]
        h_ref[...] = h

    alpha = a_ref[0, 0]
    for s in range(_S):
        o = jnp.dot(adj_ref[s], h_ref[...], preferred_element_type=jnp.float32)
        out_ref[s] = jnp.where(o >= 0, o, alpha * o)


def kernel(seq, adj, W, b, a):
    seq2 = seq.reshape(_N, _FT)
    adj3 = adj.reshape(_S, _N // _S, _N)
    b2 = b.reshape(1, _FT)
    a2 = a.reshape(1, 1)

    out = pl.pallas_call(
        _gcn_block_kernel,
        grid=(_N // _S // _BM,),
        in_specs=[
            pl.BlockSpec((_N, _FT), lambda i: (0, 0)),
            pl.BlockSpec((_FT, _FT), lambda i: (0, 0)),
            pl.BlockSpec((1, _FT), lambda i: (0, 0)),
            pl.BlockSpec((1, 1), lambda i: (0, 0)),
            pl.BlockSpec((_S, _BM, _N), lambda i: (0, i, 0)),
        ],
        out_specs=pl.BlockSpec((_S, _BM, _FT), lambda i: (0, i, 0)),
        out_shape=jax.ShapeDtypeStruct((_S, _N // _S, _FT), jnp.float32),
        scratch_shapes=[pltpu.VMEM((_N, _FT), jnp.float32)],
        compiler_params=pltpu.CompilerParams(vmem_limit_bytes=64 * 1024 * 1024),
    )(seq2, W, b2, a2, adj3)
    return out.reshape(1, _N, _FT)
